# Initial kernel scaffold; baseline (speedup 1.0000x reference)
import jax
import jax.numpy as jnp
from jax.experimental import pallas as pl


def kernel(x, edge_index, W_lin, b_lin, W_gat, att_src, att_dst, bias_gat, bn_gamma, bn_beta):
    raise NotImplementedError("write your pallas kernel here")


# speedup vs baseline: 28.1307x; 28.1307x over previous
import functools

import jax
import jax.numpy as jnp
from jax import lax
from jax.experimental import pallas as pl
from jax.experimental.pallas import tpu as pltpu
from jax.experimental.pallas import tpu_sc as plsc

_N = 10000
_E = 320000
_IN = 128
_H = 8
_D = 16
_HD = _H * _D
_TW = 144
_AW = 136
_NC = 2
_NS = 16
_NW = _NC * _NS
_EPW = _E // _NW
_C = 80
_NCH = _EPW // _C
_NP = 10240
_RPT = _NP // _NS
_ZR = 128
_BLK = 2000
_G = _N // _BLK


def _k1_body(x_ref, wg_ref, fs_ref, fd_ref, hall_ref, t2_ref):
    wg = wg_ref[...]
    fs = fs_ref[...]
    fd = fd_ref[...]
    col = lax.broadcasted_iota(jnp.int32, (_HD, _H), 0) // _D
    hid = lax.broadcasted_iota(jnp.int32, (_HD, _H), 1)
    msk = col == hid
    ms = jnp.where(msk, fs.reshape(_HD, 1), 0.0)
    md = jnp.where(msk, fd.reshape(_HD, 1), 0.0)
    vs = jnp.dot(wg, ms, preferred_element_type=jnp.float32)
    vd = jnp.dot(wg, md, preferred_element_type=jnp.float32)
    wfull = jnp.concatenate([wg, vs, vd, vd, vd], axis=1)
    out = jnp.dot(x_ref[...], wfull, preferred_element_type=jnp.float32)
    hall_ref[...] = out[:, :_TW]
    t2_ref[...] = out[:, _TW:]


def _k1(x, w_gat, fs, fd):
    return pl.pallas_call(
        _k1_body,
        grid=(_G,),
        in_specs=[
            pl.BlockSpec((_BLK, _IN), lambda i: (i, 0)),
            pl.BlockSpec((_IN, _HD), lambda i: (0, 0)),
            pl.BlockSpec((1, _HD), lambda i: (0, 0)),
            pl.BlockSpec((1, _HD), lambda i: (0, 0)),
        ],
        out_specs=[
            pl.BlockSpec((_BLK, _TW), lambda i: (i, 0)),
            pl.BlockSpec((_BLK, _D), lambda i: (i, 0)),
        ],
        out_shape=[
            jax.ShapeDtypeStruct((_N, _TW), jnp.float32),
            jax.ShapeDtypeStruct((_N, _D), jnp.float32),
        ],
    )(x, w_gat, fs, fd)


def _sc_edge(hall, t2, src, dst):
    mesh = plsc.VectorSubcoreMesh(core_axis_name="c", subcore_axis_name="s")

    @functools.partial(
        pl.kernel,
        out_type=jax.ShapeDtypeStruct((_NC, _NP, _AW), jnp.float32),
        mesh=mesh,
        compiler_params=pltpu.CompilerParams(use_tc_tiling_on_sc=False,
                                             needs_layout_passes=False),
        scratch_types=[
            pltpu.VMEM((_C,), jnp.int32),
            pltpu.VMEM((_C,), jnp.int32),
            pltpu.VMEM((_C, _TW), jnp.float32),
            pltpu.VMEM((_C, _D), jnp.float32),
            pltpu.VMEM((_C, _AW), jnp.float32),
            pltpu.VMEM((_ZR, _AW), jnp.float32),
            pltpu.VMEM_SHARED((_NP, _AW), jnp.float32),
            pltpu.SemaphoreType.DMA,
            pltpu.SemaphoreType.DMA,
        ],
    )
    def body(hall_hbm, t2_hbm, src_hbm, dst_hbm, out_hbm,
             srcv, dstv, rows, adv, msg, zbuf, acc, sem1, sem2):
        cid = lax.axis_index("c")
        sid = lax.axis_index("s")

        zcols = tuple(range(0, _HD, _D)) + (_AW - _D,)
        def zrow(r, _):
            for k in zcols:
                zbuf[r, pl.ds(k, _D)] = jnp.zeros((_D,), jnp.float32)
            return 0
        lax.fori_loop(0, _ZR, zrow, 0)
        for b in range(_RPT // _ZR):
            pltpu.sync_copy(zbuf, acc.at[pl.ds(sid * _RPT + b * _ZR, _ZR)])
        plsc.subcore_barrier()

        ebase = (cid * _NS + sid) * _EPW

        def chunk(i, _):
            base = ebase + i * _C
            pltpu.sync_copy(src_hbm.at[pl.ds(base, _C)], srcv)
            pltpu.sync_copy(dst_hbm.at[pl.ds(base, _C)], dstv)
            pltpu.async_copy(hall_hbm.at[srcv], rows, sem1).wait()
            pltpu.async_copy(t2_hbm.at[dstv], adv, sem2).wait()

            for g in range(_C // _D):
                rowi = jnp.arange(g * _D, (g + 1) * _D, dtype=jnp.int32)
                wv = []
                for h in range(_H):
                    colh = jnp.full((_D,), _HD + h, jnp.int32)
                    a1 = plsc.load_gather(rows, [rowi, colh])
                    a2 = plsc.load_gather(adv,
                                          [rowi, jnp.full((_D,), h, jnp.int32)])
                    e = a1 + a2
                    w = jnp.exp(jnp.maximum(e, 0.2 * e))
                    wv.append(w)
                    plsc.store_scatter(msg, [rowi, colh], w)
                for h in range(_H):
                    for d in range(_D):
                        col = jnp.full((_D,), h * _D + d, jnp.int32)
                        p = plsc.load_gather(rows, [rowi, col]) * wv[h]
                        plsc.store_scatter(msg, [rowi, col], p)

            pltpu.sync_copy(msg, acc.at[dstv], add=True)
            return 0

        lax.fori_loop(0, _NCH, chunk, 0)
        plsc.subcore_barrier()

        pltpu.sync_copy(acc.at[pl.ds(sid * _RPT, _RPT)],
                        out_hbm.at[cid, pl.ds(sid * _RPT, _RPT)])

    return body(hall, t2, src, dst)


def _k2a_body(a0_ref, a1_ref, bias_ref, y_ref, st_ref):
    i = pl.program_id(0)
    a = a0_ref[0] + a1_ref[0]
    wsum = a[:, _HD:_HD + _H]
    scale = 1.0 / jnp.where(wsum == 0.0, 1.0, wsum)
    col = lax.broadcasted_iota(jnp.int32, (_H, _HD), 1) // _D
    hid = lax.broadcasted_iota(jnp.int32, (_H, _HD), 0)
    rep = jnp.where(col == hid, 1.0, 0.0)
    y = a[:, :_HD] * jnp.dot(scale, rep,
                             preferred_element_type=jnp.float32)
    y = y + bias_ref[...]
    y_ref[...] = y
    st = jnp.concatenate([jnp.sum(y, axis=0, keepdims=True),
                          jnp.sum(y * y, axis=0, keepdims=True)], axis=0)

    @pl.when(i == 0)
    def _():
        st_ref[...] = st

    @pl.when(i > 0)
    def _():
        st_ref[...] += st


def _k2a(acc, bias):
    return pl.pallas_call(
        _k2a_body,
        grid=(_G,),
        in_specs=[
            pl.BlockSpec((1, _BLK, _AW), lambda i: (0, i, 0)),
            pl.BlockSpec((1, _BLK, _AW), lambda i: (1, i, 0)),
            pl.BlockSpec((1, _HD), lambda i: (0, 0)),
        ],
        out_specs=[
            pl.BlockSpec((_BLK, _HD), lambda i: (i, 0)),
            pl.BlockSpec((2, _HD), lambda i: (0, 0)),
        ],
        out_shape=[
            jax.ShapeDtypeStruct((_N, _HD), jnp.float32),
            jax.ShapeDtypeStruct((2, _HD), jnp.float32),
        ],
    )(acc, acc, bias)


def _k2b_body(y_ref, st_ref, g_ref, b_ref, o_ref):
    st = st_ref[...]
    mean = st[0:1, :] * (1.0 / _N)
    var = st[1:2, :] * (1.0 / _N) - mean * mean
    inv = lax.rsqrt(var + 1e-5) * g_ref[...]
    z = (y_ref[...] - mean) * inv + b_ref[...]
    o_ref[...] = jnp.where(z > 0, z, jnp.exp(z) - 1.0)


def _k2b(y, st, gamma, beta):
    return pl.pallas_call(
        _k2b_body,
        grid=(_G,),
        in_specs=[
            pl.BlockSpec((_BLK, _HD), lambda i: (i, 0)),
            pl.BlockSpec((2, _HD), lambda i: (0, 0)),
            pl.BlockSpec((1, _HD), lambda i: (0, 0)),
            pl.BlockSpec((1, _HD), lambda i: (0, 0)),
        ],
        out_specs=pl.BlockSpec((_BLK, _HD), lambda i: (i, 0)),
        out_shape=jax.ShapeDtypeStruct((_N, _HD), jnp.float32),
    )(y, st, gamma, beta)


def kernel(x, edge_index, W_lin, b_lin, W_gat, att_src, att_dst, bias_gat,
           bn_gamma, bn_beta):
    del W_lin, b_lin
    src = edge_index[0]
    dst = edge_index[1]
    fs = att_src.reshape(1, _HD)
    fd = att_dst.reshape(1, _HD)
    hall, t2 = _k1(x, W_gat, fs, fd)
    acc = _sc_edge(hall, t2, src, dst)
    y, st = _k2a(acc, bias_gat.reshape(1, _HD))
    return _k2b(y, st, bn_gamma.reshape(1, _HD), bn_beta.reshape(1, _HD))
